# Initial kernel scaffold; baseline (speedup 1.0000x reference)
import functools

import jax
import jax.numpy as jnp
from jax import lax
from jax.experimental import pallas as pl
from jax.experimental.pallas import tpu as pltpu
from jax.experimental.pallas import tpu_sc as plsc

NC = 2
NS = 16
LANES = 16
DEG_W = 16


def _zero_rows(ref, nrows, width):
    z = jnp.zeros((LANES,), jnp.float32)

    def body(r, _):
        for k in range(width // LANES):
            ref[r, pl.ds(k * LANES, LANES)] = z
        return 0

    lax.fori_loop(0, nrows, body, 0)


def _copy_rows(src_at, dst_at, nrows, blk):
    for off in range(0, nrows, blk):
        n = min(blk, nrows - off)
        pltpu.sync_copy(src_at(off, n), dst_at(off, n))


@functools.lru_cache(maxsize=None)
def _make_aggr(n, d, e):
    ept = e // (NC * NS)
    k = 80
    assert ept % k == 0 and e % (NC * NS) == 0
    nch = ept // k
    rows_per_tile = n // NS
    zrows = 128
    mesh = plsc.VectorSubcoreMesh(core_axis_name="c", subcore_axis_name="s")

    @functools.partial(
        pl.kernel,
        out_type=jax.ShapeDtypeStruct((NC, n, d), jnp.float32),
        mesh=mesh,
        scratch_types=[
            pltpu.VMEM_SHARED((n, d), jnp.float32),
            pltpu.VMEM((k,), jnp.int32),
            pltpu.VMEM((1, k), jnp.int32),
            pltpu.VMEM((k, d), jnp.float32),
            pltpu.VMEM((128, d), jnp.float32),
            pltpu.SemaphoreType.DMA,
        ],
    )
    def aggr(h_hbm, src_hbm, dst_hbm, out_hbm, acc_sh, src_v, dst_v, rows_v, zbuf, sem):
        c = lax.axis_index("c")
        s = lax.axis_index("s")
        wid = s * NC + c
        row0 = s * rows_per_tile

        _zero_rows(zbuf, 128, d)
        _copy_rows(lambda o, m: zbuf.at[pl.ds(0, m)],
                   lambda o, m: acc_sh.at[pl.ds(row0 + o, m)],
                   rows_per_tile, 128)
        plsc.subcore_barrier()

        base = wid * ept

        def chunk(g, _):
            off = base + g * k
            pltpu.sync_copy(src_hbm.at[pl.ds(off, k)], src_v)
            pltpu.sync_copy(dst_hbm.at[pl.ds(off, k)], dst_v.at[0])
            pltpu.async_copy(h_hbm.at[src_v], rows_v, sem).wait()
            pltpu.sync_copy(rows_v, acc_sh.at[dst_v.at[0]], add=True)
            return 0

        lax.fori_loop(0, nch, chunk, 0)
        plsc.subcore_barrier()

        _copy_rows(lambda o, m: acc_sh.at[pl.ds(row0 + o, m)],
                   lambda o, m: out_hbm.at[c, pl.ds(row0 + o, m)],
                   rows_per_tile, 128)

    return aggr


@functools.lru_cache(maxsize=None)
def _make_deg(n, e):
    ept = e // (NC * NS)
    k = 80
    assert ept % k == 0
    nch = ept // k
    rows_per_tile = n // NS
    mesh = plsc.VectorSubcoreMesh(core_axis_name="c", subcore_axis_name="s")

    @functools.partial(
        pl.kernel,
        out_type=jax.ShapeDtypeStruct((NC, n, DEG_W), jnp.float32),
        mesh=mesh,
        scratch_types=[
            pltpu.VMEM_SHARED((n, DEG_W), jnp.float32),
            pltpu.VMEM((1, k), jnp.int32),
            pltpu.VMEM((k, DEG_W), jnp.float32),
            pltpu.VMEM((128, DEG_W), jnp.float32),
        ],
    )
    def deg(dst_hbm, out_hbm, acc_sh, dst_v, ones_v, zbuf):
        c = lax.axis_index("c")
        s = lax.axis_index("s")
        wid = s * NC + c
        row0 = s * rows_per_tile

        one = jnp.ones((LANES,), jnp.float32)

        def fill(r, _):
            ones_v[r, :] = one
            return 0

        lax.fori_loop(0, k, fill, 0)
        _zero_rows(zbuf, 128, DEG_W)
        _copy_rows(lambda o, m: zbuf.at[pl.ds(0, m)],
                   lambda o, m: acc_sh.at[pl.ds(row0 + o, m)],
                   rows_per_tile, 128)
        plsc.subcore_barrier()

        base = wid * ept

        def chunk(g, _):
            off = base + g * k
            pltpu.sync_copy(dst_hbm.at[pl.ds(off, k)], dst_v.at[0])
            pltpu.sync_copy(ones_v, acc_sh.at[dst_v.at[0]], add=True)
            return 0

        lax.fori_loop(0, nch, chunk, 0)
        plsc.subcore_barrier()

        _copy_rows(lambda o, m: acc_sh.at[pl.ds(row0 + o, m)],
                   lambda o, m: out_hbm.at[c, pl.ds(row0 + o, m)],
                   rows_per_tile, 128)

    return deg


def _emb_body(x_ref, w_ref, b_ref, o_ref):
    y = lax.dot_general(x_ref[...], w_ref[...], (((1,), (1,)), ((), ())),
                        preferred_element_type=jnp.float32)
    o_ref[...] = jnp.maximum(y + b_ref[...], 0.0)


@functools.lru_cache(maxsize=None)
def _make_emb(n, d, bn):
    return pl.pallas_call(
        _emb_body,
        grid=(n // bn,),
        in_specs=[
            pl.BlockSpec((bn, d), lambda i: (i, 0)),
            pl.BlockSpec((d, d), lambda i: (0, 0)),
            pl.BlockSpec((1, d), lambda i: (0, 0)),
        ],
        out_specs=pl.BlockSpec((bn, d), lambda i: (i, 0)),
        out_shape=jax.ShapeDtypeStruct((n, d), jnp.float32),
    )


def _layer_body(p_ref, dg_ref, h_ref, wl_ref, bl_ref, wr_ref, o_ref):
    ssum = p_ref[0] + p_ref[1]
    dsum = dg_ref[0] + dg_ref[1]
    degc = jnp.max(dsum, axis=1, keepdims=True)
    aggr = ssum / jnp.maximum(degc, 1.0)
    y = lax.dot_general(aggr, wl_ref[...], (((1,), (1,)), ((), ())),
                        preferred_element_type=jnp.float32)
    y2 = lax.dot_general(h_ref[...], wr_ref[...], (((1,), (1,)), ((), ())),
                         preferred_element_type=jnp.float32)
    o_ref[...] = jnp.maximum(y + bl_ref[...] + y2, 0.0)


@functools.lru_cache(maxsize=None)
def _make_layer(n, d, bn):
    return pl.pallas_call(
        _layer_body,
        grid=(n // bn,),
        in_specs=[
            pl.BlockSpec((NC, bn, d), lambda i: (0, i, 0)),
            pl.BlockSpec((NC, bn, DEG_W), lambda i: (0, i, 0)),
            pl.BlockSpec((bn, d), lambda i: (i, 0)),
            pl.BlockSpec((d, d), lambda i: (0, 0)),
            pl.BlockSpec((1, d), lambda i: (0, 0)),
            pl.BlockSpec((d, d), lambda i: (0, 0)),
        ],
        out_specs=pl.BlockSpec((bn, d), lambda i: (i, 0)),
        out_shape=jax.ShapeDtypeStruct((n, d), jnp.float32),
    )


def kernel(x, edge_index, W_emb, b_emb,
           W_l0, b_l0, W_r0,
           W_l1, b_l1, W_r1,
           W_l2, b_l2, W_r2,
           W_l3, b_l3, W_r3):
    n, d = x.shape
    e = edge_index.shape[1]
    src = edge_index[0]
    dst = edge_index[1]

    bn = 2000
    emb = _make_emb(n, d, bn)
    layer = _make_layer(n, d, bn)
    aggr = _make_aggr(n, d, e)
    deg = _make_deg(n, e)

    h = emb(x, W_emb, b_emb.reshape(1, d))
    degp = deg(dst)
    for wl, bl, wr in ((W_l0, b_l0, W_r0), (W_l1, b_l1, W_r1),
                       (W_l2, b_l2, W_r2), (W_l3, b_l3, W_r3)):
        p = aggr(h, src, dst)
        h = layer(p, degp, h, wl, bl.reshape(1, d), wr)
    return h


# speedup vs baseline: 4.4181x; 4.4181x over previous
import functools

import jax
import jax.numpy as jnp
from jax import lax
from jax.experimental import pallas as pl
from jax.experimental.pallas import tpu as pltpu
from jax.experimental.pallas import tpu_sc as plsc

NC = 2
NS = 16
LANES = 16
DEG_W = 128


def _fill_rows(ref, nrows, width, const):
    v = jnp.full((LANES,), const, jnp.float32)

    def body(r, _):
        for k in range(width // LANES):
            ref[r, pl.ds(k * LANES, LANES)] = v
        return 0

    lax.fori_loop(0, nrows, body, 0)


def _zero_rows(ref, nrows, width):
    _fill_rows(ref, nrows, width, 0.0)


def _copy_rows(src_at, dst_at, nrows, blk):
    for off in range(0, nrows, blk):
        n = min(blk, nrows - off)
        pltpu.sync_copy(src_at(off, n), dst_at(off, n))


@functools.lru_cache(maxsize=None)
def _make_aggr(n, d, e):
    ept = e // (NC * NS)
    k = 80
    assert ept % k == 0 and e % (NC * NS) == 0
    nch = ept // k
    rpt = (n // NS) // 8 * 8
    rem = n - rpt * NS
    assert rem % 8 == 0
    mesh = plsc.VectorSubcoreMesh(core_axis_name="c", subcore_axis_name="s")

    @functools.partial(
        pl.kernel,
        out_type=jax.ShapeDtypeStruct((NC, n, d), jnp.float32),
        mesh=mesh,
        scratch_types=[
            pltpu.VMEM_SHARED((n, d), jnp.float32),
            pltpu.VMEM((k,), jnp.int32),
            pltpu.VMEM((1, k), jnp.int32),
            pltpu.VMEM((k, d), jnp.float32),
            pltpu.VMEM((128, d), jnp.float32),
            pltpu.SemaphoreType.DMA,
        ],
    )
    def aggr(h_hbm, src_hbm, dst_hbm, out_hbm, acc_sh, src_v, dst_v, rows_v, zbuf, sem):
        c = lax.axis_index("c")
        s = lax.axis_index("s")
        wid = s * NC + c
        row0 = s * rpt

        _zero_rows(zbuf, 128, d)
        _copy_rows(lambda o, m: zbuf.at[pl.ds(0, m)],
                   lambda o, m: acc_sh.at[pl.ds(row0 + o, m)],
                   rpt, 128)
        if rem:
            @pl.when(s == NS - 1)
            def _():
                _copy_rows(lambda o, m: zbuf.at[pl.ds(0, m)],
                           lambda o, m: acc_sh.at[pl.ds(rpt * NS + o, m)],
                           rem, 128)
        plsc.subcore_barrier()

        base = wid * ept

        def chunk(g, _):
            off = base + g * k
            pltpu.sync_copy(src_hbm.at[pl.ds(off, k)], src_v)
            pltpu.sync_copy(dst_hbm.at[pl.ds(off, k)], dst_v.at[0])
            pltpu.async_copy(h_hbm.at[src_v], rows_v, sem).wait()
            pltpu.sync_copy(rows_v, acc_sh.at[dst_v.at[0]], add=True)
            return 0

        lax.fori_loop(0, nch, chunk, 0)
        plsc.subcore_barrier()

        _copy_rows(lambda o, m: acc_sh.at[pl.ds(row0 + o, m)],
                   lambda o, m: out_hbm.at[c, pl.ds(row0 + o, m)],
                   rpt, 128)
        if rem:
            @pl.when(s == NS - 1)
            def _():
                _copy_rows(lambda o, m: acc_sh.at[pl.ds(rpt * NS + o, m)],
                           lambda o, m: out_hbm.at[c, pl.ds(rpt * NS + o, m)],
                           rem, 128)

    return aggr


@functools.lru_cache(maxsize=None)
def _make_deg(n, e, deg_w=DEG_W):
    DW = deg_w
    ept = e // (NC * NS)
    k = 80
    assert ept % k == 0
    nch = ept // k
    rpt = (n // NS) // 8 * 8
    rem = n - rpt * NS
    assert rem % 8 == 0
    mesh = plsc.VectorSubcoreMesh(core_axis_name="c", subcore_axis_name="s")

    @functools.partial(
        pl.kernel,
        out_type=jax.ShapeDtypeStruct((NC, n, DW), jnp.float32),
        mesh=mesh,
        scratch_types=[
            pltpu.VMEM_SHARED((n, DW), jnp.float32),
            pltpu.VMEM((1, k), jnp.int32),
            pltpu.VMEM((k, DW), jnp.float32),
            pltpu.VMEM((128, DW), jnp.float32),
        ],
    )
    def deg(dst_hbm, out_hbm, acc_sh, dst_v, ones_v, zbuf):
        c = lax.axis_index("c")
        s = lax.axis_index("s")
        wid = s * NC + c
        row0 = s * rpt

        _fill_rows(ones_v, k, DW, 1.0)
        _zero_rows(zbuf, 128, DW)
        _copy_rows(lambda o, m: zbuf.at[pl.ds(0, m)],
                   lambda o, m: acc_sh.at[pl.ds(row0 + o, m)],
                   rpt, 128)
        if rem:
            @pl.when(s == NS - 1)
            def _():
                _copy_rows(lambda o, m: zbuf.at[pl.ds(0, m)],
                           lambda o, m: acc_sh.at[pl.ds(rpt * NS + o, m)],
                           rem, 128)
        plsc.subcore_barrier()

        base = wid * ept

        def chunk(g, _):
            off = base + g * k
            pltpu.sync_copy(dst_hbm.at[pl.ds(off, k)], dst_v.at[0])
            pltpu.sync_copy(ones_v, acc_sh.at[dst_v.at[0]], add=True)
            return 0

        lax.fori_loop(0, nch, chunk, 0)
        plsc.subcore_barrier()

        _copy_rows(lambda o, m: acc_sh.at[pl.ds(row0 + o, m)],
                   lambda o, m: out_hbm.at[c, pl.ds(row0 + o, m)],
                   rpt, 128)
        if rem:
            @pl.when(s == NS - 1)
            def _():
                _copy_rows(lambda o, m: acc_sh.at[pl.ds(rpt * NS + o, m)],
                           lambda o, m: out_hbm.at[c, pl.ds(rpt * NS + o, m)],
                           rem, 128)

    return deg


def _emb_body(x_ref, w_ref, b_ref, o_ref):
    y = lax.dot_general(x_ref[...], w_ref[...], (((1,), (1,)), ((), ())),
                        preferred_element_type=jnp.float32)
    o_ref[...] = jnp.maximum(y + b_ref[...], 0.0)


@functools.lru_cache(maxsize=None)
def _make_emb(n, d, bn):
    return pl.pallas_call(
        _emb_body,
        grid=(n // bn,),
        in_specs=[
            pl.BlockSpec((bn, d), lambda i: (i, 0)),
            pl.BlockSpec((d, d), lambda i: (0, 0)),
            pl.BlockSpec((1, d), lambda i: (0, 0)),
        ],
        out_specs=pl.BlockSpec((bn, d), lambda i: (i, 0)),
        out_shape=jax.ShapeDtypeStruct((n, d), jnp.float32),
    )


def _layer_body(p_ref, dg_ref, h_ref, wl_ref, bl_ref, wr_ref, o_ref):
    ssum = p_ref[0] + p_ref[1]
    dsum = dg_ref[0] + dg_ref[1]
    aggr = ssum / jnp.maximum(dsum, 1.0)
    y = lax.dot_general(aggr, wl_ref[...], (((1,), (1,)), ((), ())),
                        preferred_element_type=jnp.float32)
    y2 = lax.dot_general(h_ref[...], wr_ref[...], (((1,), (1,)), ((), ())),
                         preferred_element_type=jnp.float32)
    o_ref[...] = jnp.maximum(y + bl_ref[...] + y2, 0.0)


@functools.lru_cache(maxsize=None)
def _make_layer(n, d, bn):
    return pl.pallas_call(
        _layer_body,
        grid=(n // bn,),
        in_specs=[
            pl.BlockSpec((NC, bn, d), lambda i: (0, i, 0)),
            pl.BlockSpec((NC, bn, DEG_W), lambda i: (0, i, 0)),
            pl.BlockSpec((bn, d), lambda i: (i, 0)),
            pl.BlockSpec((d, d), lambda i: (0, 0)),
            pl.BlockSpec((1, d), lambda i: (0, 0)),
            pl.BlockSpec((d, d), lambda i: (0, 0)),
        ],
        out_specs=pl.BlockSpec((bn, d), lambda i: (i, 0)),
        out_shape=jax.ShapeDtypeStruct((n, d), jnp.float32),
    )


def kernel(x, edge_index, W_emb, b_emb,
           W_l0, b_l0, W_r0,
           W_l1, b_l1, W_r1,
           W_l2, b_l2, W_r2,
           W_l3, b_l3, W_r3):
    n, d = x.shape
    e = edge_index.shape[1]
    src = edge_index[0]
    dst = edge_index[1]

    bn = 2000
    emb = _make_emb(n, d, bn)
    layer = _make_layer(n, d, bn)
    aggr = _make_aggr(n, d, e)
    deg = _make_deg(n, e)

    h = emb(x, W_emb, b_emb.reshape(1, d))
    degp = deg(dst)
    for wl, bl, wr in ((W_l0, b_l0, W_r0), (W_l1, b_l1, W_r1),
                       (W_l2, b_l2, W_r2), (W_l3, b_l3, W_r3)):
        p = aggr(h, src, dst)
        h = layer(p, degp, h, wl, bl.reshape(1, d), wr)
    return h


# speedup vs baseline: 11.3758x; 2.5748x over previous
import functools

import jax
import jax.numpy as jnp
from jax import lax
from jax.experimental import pallas as pl
from jax.experimental.pallas import tpu as pltpu
from jax.experimental.pallas import tpu_sc as plsc

NC = 2
NS = 16
LANES = 16
DEG_W = 128


def _fill_rows(ref, nrows, width, const):
    v = jnp.full((LANES,), const, jnp.float32)

    def body(r, _):
        for k in range(width // LANES):
            ref[r, pl.ds(k * LANES, LANES)] = v
        return 0

    lax.fori_loop(0, nrows, body, 0)


def _zero_rows(ref, nrows, width):
    _fill_rows(ref, nrows, width, 0.0)


def _copy_rows(src_at, dst_at, nrows, blk):
    for off in range(0, nrows, blk):
        n = min(blk, nrows - off)
        pltpu.sync_copy(src_at(off, n), dst_at(off, n))


NBUF = 3
NIDX = 6


@functools.lru_cache(maxsize=None)
def _make_aggr(n, d, e):
    ept = e // (NC * NS)
    k = 80
    assert ept % k == 0 and e % (NC * NS) == 0
    nch = ept // k
    rpt = (n // NS) // 8 * 8
    rem = n - rpt * NS
    assert rem % 8 == 0
    mesh = plsc.VectorSubcoreMesh(core_axis_name="c", subcore_axis_name="s")

    @functools.partial(
        pl.kernel,
        out_type=jax.ShapeDtypeStruct((NC, n, d), jnp.float32),
        mesh=mesh,
        scratch_types=[
            pltpu.VMEM_SHARED((n, d), jnp.float32),
            pltpu.VMEM((NIDX, 2, k), jnp.int32),
            pltpu.VMEM((NBUF * k, d), jnp.float32),
        ] + [pltpu.SemaphoreType.DMA] * (2 * NBUF + NIDX),
    )
    def aggr(h_hbm, idx_hbm, out_hbm, acc_sh, idx_t, rows, *sems):
        isem = sems[:NIDX]
        gsem = sems[NIDX:NIDX + NBUF]
        ssem = sems[NIDX + NBUF:]
        c = lax.axis_index("c")
        s = lax.axis_index("s")
        wid = s * NC + c
        row0 = s * rpt

        buf = [rows.at[pl.ds(b * k, k)] for b in range(NBUF)]

        def i_start(g, sl):
            pltpu.async_copy(idx_hbm.at[wid, g], idx_t.at[sl], isem[sl])

        def i_wait(g, sl):
            pltpu.make_async_copy(idx_hbm.at[wid, g], idx_t.at[sl], isem[sl]).wait()

        def g_start(b, sl):
            pltpu.async_copy(h_hbm.at[idx_t.at[sl, 0]], buf[b], gsem[b])

        def g_wait(b, sl):
            pltpu.make_async_copy(h_hbm.at[idx_t.at[sl, 0]], buf[b], gsem[b]).wait()

        def s_start(b, sl):
            pltpu.async_copy(buf[b], acc_sh.at[idx_t.at[sl, 1]], ssem[b], add=True)

        def s_wait(b, sl):
            pltpu.make_async_copy(buf[b], acc_sh.at[idx_t.at[sl, 1]], ssem[b]).wait()

        zbuf = rows.at[pl.ds(0, 128)]
        _zero_rows(rows, 128, d)
        _copy_rows(lambda o, m: zbuf.at[pl.ds(0, m)],
                   lambda o, m: acc_sh.at[pl.ds(row0 + o, m)],
                   rpt, 128)
        if rem:
            @pl.when(s == NS - 1)
            def _():
                _copy_rows(lambda o, m: zbuf.at[pl.ds(0, m)],
                           lambda o, m: acc_sh.at[pl.ds(rpt * NS + o, m)],
                           rem, 128)
        plsc.subcore_barrier()

        for sl in range(NIDX):
            i_start(sl, sl)
        i_wait(0, 0)
        g_start(0, 0)
        i_wait(1, 1)
        g_start(1, 1)

        ngrp = nch // NIDX
        tail = nch - ngrp * NIDX

        def step_main(g, j):
            b, sl = j % NBUF, j % NIDX
            bn_, sln_ = (b + NBUF - 1) % NBUF, (sl + 2) % NIDX
            slp_ = (sl + NIDX - 1) % NIDX
            g_wait(b, sl)

            def drain_and_prefetch():
                s_wait(bn_, slp_)
                i_start(g + NIDX - 1, slp_)
            if j == 0:
                pl.when(g > 0)(drain_and_prefetch)
            else:
                drain_and_prefetch()
            i_wait(g + 2, sln_)
            g_start(bn_, sln_)
            s_start(b, sl)

        def group(i, _):
            for j in range(NIDX):
                step_main(i * NIDX + j, j)
            return 0

        lax.fori_loop(0, ngrp, group, 0)
        for t in range(tail):
            g = ngrp * NIDX + t
            b, sl = g % NBUF, g % NIDX
            bn_, sln_ = (b + NBUF - 1) % NBUF, (sl + 2) % NIDX
            slp_ = (sl + NIDX - 1) % NIDX
            g_wait(b, sl)
            s_wait(bn_, slp_)
            if g + 2 < nch:
                i_wait(g + 2, sln_)
                g_start(bn_, sln_)
            s_start(b, sl)
        s_wait((nch - 1) % NBUF, (nch - 1) % NIDX)
        plsc.subcore_barrier()

        _copy_rows(lambda o, m: acc_sh.at[pl.ds(row0 + o, m)],
                   lambda o, m: out_hbm.at[c, pl.ds(row0 + o, m)],
                   rpt, 128)
        if rem:
            @pl.when(s == NS - 1)
            def _():
                _copy_rows(lambda o, m: acc_sh.at[pl.ds(rpt * NS + o, m)],
                           lambda o, m: out_hbm.at[c, pl.ds(rpt * NS + o, m)],
                           rem, 128)

    return aggr


@functools.lru_cache(maxsize=None)
def _make_deg(n, e, deg_w=DEG_W):
    DW = deg_w
    ept = e // (NC * NS)
    k = 80
    assert ept % k == 0
    nch = ept // k
    rpt = (n // NS) // 8 * 8
    rem = n - rpt * NS
    assert rem % 8 == 0
    mesh = plsc.VectorSubcoreMesh(core_axis_name="c", subcore_axis_name="s")

    @functools.partial(
        pl.kernel,
        out_type=jax.ShapeDtypeStruct((NC, n, DW), jnp.float32),
        mesh=mesh,
        scratch_types=[
            pltpu.VMEM_SHARED((n, DW), jnp.float32),
            pltpu.VMEM((nch, k), jnp.int32),
            pltpu.VMEM((k, DW), jnp.float32),
            pltpu.VMEM((64, DW), jnp.float32),
        ] + [pltpu.SemaphoreType.DMA] * NBUF,
    )
    def deg(dst_hbm, out_hbm, acc_sh, dst_t, ones_v, zbuf, *ssem):
        c = lax.axis_index("c")
        s = lax.axis_index("s")
        wid = s * NC + c
        row0 = s * rpt

        def s_start(g, b):
            pltpu.async_copy(ones_v, acc_sh.at[dst_t.at[g]], ssem[b], add=True)

        def s_wait(g, b):
            pltpu.make_async_copy(ones_v, acc_sh.at[dst_t.at[g]], ssem[b]).wait()

        pltpu.sync_copy(dst_hbm.at[wid], dst_t)
        _fill_rows(ones_v, k, DW, 1.0)
        _zero_rows(zbuf, 64, DW)
        _copy_rows(lambda o, m: zbuf.at[pl.ds(0, m)],
                   lambda o, m: acc_sh.at[pl.ds(row0 + o, m)],
                   rpt, 64)
        if rem:
            @pl.when(s == NS - 1)
            def _():
                _copy_rows(lambda o, m: zbuf.at[pl.ds(0, m)],
                           lambda o, m: acc_sh.at[pl.ds(rpt * NS + o, m)],
                           rem, 64)
        plsc.subcore_barrier()

        for b in range(min(NBUF, nch)):
            s_start(b, b)
        ngrp = nch // NBUF
        tail = nch - ngrp * NBUF

        def group(i, _):
            for j in range(NBUF):
                g = i * NBUF + NBUF + j
                s_wait(g - NBUF, j)
                s_start(g, j)
            return 0

        lax.fori_loop(0, ngrp - 1, group, 0)
        for j in range(tail):
            g = (ngrp - 1) * NBUF + NBUF + j
            s_wait(g - NBUF, j)
            s_start(g, j)
        for g in range(nch - NBUF, nch):
            s_wait(g, g % NBUF)
        plsc.subcore_barrier()

        _copy_rows(lambda o, m: acc_sh.at[pl.ds(row0 + o, m)],
                   lambda o, m: out_hbm.at[c, pl.ds(row0 + o, m)],
                   rpt, 128)
        if rem:
            @pl.when(s == NS - 1)
            def _():
                _copy_rows(lambda o, m: acc_sh.at[pl.ds(rpt * NS + o, m)],
                           lambda o, m: out_hbm.at[c, pl.ds(rpt * NS + o, m)],
                           rem, 128)

    return deg


def _emb_body(x_ref, w_ref, b_ref, o_ref):
    y = lax.dot_general(x_ref[...], w_ref[...], (((1,), (1,)), ((), ())),
                        preferred_element_type=jnp.float32)
    o_ref[...] = jnp.maximum(y + b_ref[...], 0.0)


@functools.lru_cache(maxsize=None)
def _make_emb(n, d, bn):
    return pl.pallas_call(
        _emb_body,
        grid=(n // bn,),
        in_specs=[
            pl.BlockSpec((bn, d), lambda i: (i, 0)),
            pl.BlockSpec((d, d), lambda i: (0, 0)),
            pl.BlockSpec((1, d), lambda i: (0, 0)),
        ],
        out_specs=pl.BlockSpec((bn, d), lambda i: (i, 0)),
        out_shape=jax.ShapeDtypeStruct((n, d), jnp.float32),
    )


def _layer_body(p_ref, dg_ref, h_ref, wl_ref, bl_ref, wr_ref, o_ref):
    ssum = p_ref[0] + p_ref[1]
    dsum = dg_ref[0] + dg_ref[1]
    aggr = ssum / jnp.maximum(dsum, 1.0)
    y = lax.dot_general(aggr, wl_ref[...], (((1,), (1,)), ((), ())),
                        preferred_element_type=jnp.float32)
    y2 = lax.dot_general(h_ref[...], wr_ref[...], (((1,), (1,)), ((), ())),
                         preferred_element_type=jnp.float32)
    o_ref[...] = jnp.maximum(y + bl_ref[...] + y2, 0.0)


@functools.lru_cache(maxsize=None)
def _make_layer(n, d, bn):
    return pl.pallas_call(
        _layer_body,
        grid=(n // bn,),
        in_specs=[
            pl.BlockSpec((NC, bn, d), lambda i: (0, i, 0)),
            pl.BlockSpec((NC, bn, DEG_W), lambda i: (0, i, 0)),
            pl.BlockSpec((bn, d), lambda i: (i, 0)),
            pl.BlockSpec((d, d), lambda i: (0, 0)),
            pl.BlockSpec((1, d), lambda i: (0, 0)),
            pl.BlockSpec((d, d), lambda i: (0, 0)),
        ],
        out_specs=pl.BlockSpec((bn, d), lambda i: (i, 0)),
        out_shape=jax.ShapeDtypeStruct((n, d), jnp.float32),
    )


def kernel(x, edge_index, W_emb, b_emb,
           W_l0, b_l0, W_r0,
           W_l1, b_l1, W_r1,
           W_l2, b_l2, W_r2,
           W_l3, b_l3, W_r3):
    n, d = x.shape
    e = edge_index.shape[1]
    k = 80
    nch = e // (NC * NS * k)
    src3 = edge_index[0].reshape(NC * NS, nch, k)
    dst3 = edge_index[1].reshape(NC * NS, nch, k)
    idx2 = jnp.stack([src3, dst3], axis=2)

    bn = 2000
    emb = _make_emb(n, d, bn)
    layer = _make_layer(n, d, bn)
    aggr = _make_aggr(n, d, e)
    deg = _make_deg(n, e)

    h = emb(x, W_emb, b_emb.reshape(1, d))
    degp = deg(dst3)
    for wl, bl, wr in ((W_l0, b_l0, W_r0), (W_l1, b_l1, W_r1),
                       (W_l2, b_l2, W_r2), (W_l3, b_l3, W_r3)):
        p = aggr(h, idx2)
        h = layer(p, degp, h, wl, bl.reshape(1, d), wr)
    return h


# speedup vs baseline: 11.5929x; 1.0191x over previous
import functools

import jax
import jax.numpy as jnp
from jax import lax
from jax.experimental import pallas as pl
from jax.experimental.pallas import tpu as pltpu
from jax.experimental.pallas import tpu_sc as plsc

NC = 2
NS = 16
LANES = 16
DEG_W = 128


def _fill_rows(ref, nrows, width, const):
    v = jnp.full((LANES,), const, jnp.float32)

    def body(r, _):
        for k in range(width // LANES):
            ref[r, pl.ds(k * LANES, LANES)] = v
        return 0

    lax.fori_loop(0, nrows, body, 0)


def _zero_rows(ref, nrows, width):
    _fill_rows(ref, nrows, width, 0.0)


def _copy_rows(src_at, dst_at, nrows, blk):
    for off in range(0, nrows, blk):
        n = min(blk, nrows - off)
        pltpu.sync_copy(src_at(off, n), dst_at(off, n))


NBUF = 4
NIDX = 8
LOOK = NBUF - 1
KAGG = 40
KDEG = 80


@functools.lru_cache(maxsize=None)
def _make_aggr(n, d, e):
    ept = e // (NC * NS)
    k = KAGG
    assert ept % k == 0 and e % (NC * NS) == 0
    nch = ept // k
    rpt = (n // NS) // 8 * 8
    rem = n - rpt * NS
    assert rem % 8 == 0
    mesh = plsc.VectorSubcoreMesh(core_axis_name="c", subcore_axis_name="s")

    @functools.partial(
        pl.kernel,
        out_type=jax.ShapeDtypeStruct((NC, n, d), jnp.float32),
        mesh=mesh,
        scratch_types=[
            pltpu.VMEM_SHARED((n, d), jnp.float32),
            pltpu.VMEM((NIDX, 2, k), jnp.int32),
            pltpu.VMEM((NBUF * k, d), jnp.float32),
        ] + [pltpu.SemaphoreType.DMA] * (2 * NBUF + NIDX),
    )
    def aggr(h_hbm, idx_hbm, out_hbm, acc_sh, idx_t, rows, *sems):
        isem = sems[:NIDX]
        gsem = sems[NIDX:NIDX + NBUF]
        ssem = sems[NIDX + NBUF:]
        c = lax.axis_index("c")
        s = lax.axis_index("s")
        wid = s * NC + c
        row0 = s * rpt

        buf = [rows.at[pl.ds(b * k, k)] for b in range(NBUF)]

        def i_start(g, sl):
            pltpu.async_copy(idx_hbm.at[wid, g], idx_t.at[sl], isem[sl])

        def i_wait(g, sl):
            pltpu.make_async_copy(idx_hbm.at[wid, g], idx_t.at[sl], isem[sl]).wait()

        def g_start(b, sl):
            pltpu.async_copy(h_hbm.at[idx_t.at[sl, 0]], buf[b], gsem[b])

        def g_wait(b, sl):
            pltpu.make_async_copy(h_hbm.at[idx_t.at[sl, 0]], buf[b], gsem[b]).wait()

        def s_start(b, sl):
            pltpu.async_copy(buf[b], acc_sh.at[idx_t.at[sl, 1]], ssem[b], add=True)

        def s_wait(b, sl):
            pltpu.make_async_copy(buf[b], acc_sh.at[idx_t.at[sl, 1]], ssem[b]).wait()

        zbuf = rows.at[pl.ds(0, 128)]
        _zero_rows(rows, 128, d)
        _copy_rows(lambda o, m: zbuf.at[pl.ds(0, m)],
                   lambda o, m: acc_sh.at[pl.ds(row0 + o, m)],
                   rpt, 128)
        if rem:
            @pl.when(s == NS - 1)
            def _():
                _copy_rows(lambda o, m: zbuf.at[pl.ds(0, m)],
                           lambda o, m: acc_sh.at[pl.ds(rpt * NS + o, m)],
                           rem, 128)
        plsc.subcore_barrier()

        for sl in range(NIDX):
            i_start(sl, sl)
        for b in range(LOOK):
            i_wait(b, b)
            g_start(b, b)

        ngrp = nch // NIDX
        tail = nch - ngrp * NIDX

        def step_main(g, j):
            b, sl = j % NBUF, j % NIDX
            bn_, sln_ = (b + LOOK) % NBUF, (sl + LOOK) % NIDX
            slp_ = (sl + NIDX - 1) % NIDX
            g_wait(b, sl)

            def drain_and_prefetch():
                s_wait(bn_, slp_)

                @pl.when(g + NIDX - 1 < nch)
                def _():
                    i_start(g + NIDX - 1, slp_)
            if j == 0:
                pl.when(g > 0)(drain_and_prefetch)
            else:
                drain_and_prefetch()

            @pl.when(g + LOOK < nch)
            def _():
                i_wait(g + LOOK, sln_)
                g_start(bn_, sln_)
            s_start(b, sl)

        def group(i, _):
            for j in range(NIDX):
                step_main(i * NIDX + j, j)
            return 0

        lax.fori_loop(0, ngrp, group, 0)
        for t in range(tail):
            g = ngrp * NIDX + t
            b, sl = g % NBUF, g % NIDX
            bn_, sln_ = (b + LOOK) % NBUF, (sl + LOOK) % NIDX
            slp_ = (sl + NIDX - 1) % NIDX
            g_wait(b, sl)
            s_wait(bn_, slp_)
            if g + LOOK < nch:
                i_wait(g + LOOK, sln_)
                g_start(bn_, sln_)
            s_start(b, sl)
        s_wait((nch - 1) % NBUF, (nch - 1) % NIDX)
        plsc.subcore_barrier()

        _copy_rows(lambda o, m: acc_sh.at[pl.ds(row0 + o, m)],
                   lambda o, m: out_hbm.at[c, pl.ds(row0 + o, m)],
                   rpt, 128)
        if rem:
            @pl.when(s == NS - 1)
            def _():
                _copy_rows(lambda o, m: acc_sh.at[pl.ds(rpt * NS + o, m)],
                           lambda o, m: out_hbm.at[c, pl.ds(rpt * NS + o, m)],
                           rem, 128)

    return aggr


@functools.lru_cache(maxsize=None)
def _make_deg(n, e, deg_w=DEG_W):
    DW = deg_w
    ept = e // (NC * NS)
    k = KDEG
    assert ept % k == 0
    nch = ept // k
    rpt = (n // NS) // 8 * 8
    rem = n - rpt * NS
    assert rem % 8 == 0
    mesh = plsc.VectorSubcoreMesh(core_axis_name="c", subcore_axis_name="s")

    @functools.partial(
        pl.kernel,
        out_type=jax.ShapeDtypeStruct((NC, n, DW), jnp.float32),
        mesh=mesh,
        scratch_types=[
            pltpu.VMEM_SHARED((n, DW), jnp.float32),
            pltpu.VMEM((nch, k), jnp.int32),
            pltpu.VMEM((k, DW), jnp.float32),
            pltpu.VMEM((64, DW), jnp.float32),
        ] + [pltpu.SemaphoreType.DMA] * NBUF,
    )
    def deg(dst_hbm, out_hbm, acc_sh, dst_t, ones_v, zbuf, *ssem):
        c = lax.axis_index("c")
        s = lax.axis_index("s")
        wid = s * NC + c
        row0 = s * rpt

        def s_start(g, b):
            pltpu.async_copy(ones_v, acc_sh.at[dst_t.at[g]], ssem[b], add=True)

        def s_wait(g, b):
            pltpu.make_async_copy(ones_v, acc_sh.at[dst_t.at[g]], ssem[b]).wait()

        pltpu.sync_copy(dst_hbm.at[wid], dst_t)
        _fill_rows(ones_v, k, DW, 1.0)
        _zero_rows(zbuf, 64, DW)
        _copy_rows(lambda o, m: zbuf.at[pl.ds(0, m)],
                   lambda o, m: acc_sh.at[pl.ds(row0 + o, m)],
                   rpt, 64)
        if rem:
            @pl.when(s == NS - 1)
            def _():
                _copy_rows(lambda o, m: zbuf.at[pl.ds(0, m)],
                           lambda o, m: acc_sh.at[pl.ds(rpt * NS + o, m)],
                           rem, 64)
        plsc.subcore_barrier()

        for b in range(min(NBUF, nch)):
            s_start(b, b)
        ngrp = nch // NBUF
        tail = nch - ngrp * NBUF

        def group(i, _):
            for j in range(NBUF):
                g = i * NBUF + NBUF + j
                s_wait(g - NBUF, j)
                s_start(g, j)
            return 0

        lax.fori_loop(0, ngrp - 1, group, 0)
        for j in range(tail):
            g = (ngrp - 1) * NBUF + NBUF + j
            s_wait(g - NBUF, j)
            s_start(g, j)
        for g in range(nch - NBUF, nch):
            s_wait(g, g % NBUF)
        plsc.subcore_barrier()

        _copy_rows(lambda o, m: acc_sh.at[pl.ds(row0 + o, m)],
                   lambda o, m: out_hbm.at[c, pl.ds(row0 + o, m)],
                   rpt, 128)
        if rem:
            @pl.when(s == NS - 1)
            def _():
                _copy_rows(lambda o, m: acc_sh.at[pl.ds(rpt * NS + o, m)],
                           lambda o, m: out_hbm.at[c, pl.ds(rpt * NS + o, m)],
                           rem, 128)

    return deg


def _emb_body(x_ref, w_ref, b_ref, o_ref):
    y = lax.dot_general(x_ref[...], w_ref[...], (((1,), (1,)), ((), ())),
                        preferred_element_type=jnp.float32)
    o_ref[...] = jnp.maximum(y + b_ref[...], 0.0)


@functools.lru_cache(maxsize=None)
def _make_emb(n, d, bn):
    return pl.pallas_call(
        _emb_body,
        grid=(n // bn,),
        in_specs=[
            pl.BlockSpec((bn, d), lambda i: (i, 0)),
            pl.BlockSpec((d, d), lambda i: (0, 0)),
            pl.BlockSpec((1, d), lambda i: (0, 0)),
        ],
        out_specs=pl.BlockSpec((bn, d), lambda i: (i, 0)),
        out_shape=jax.ShapeDtypeStruct((n, d), jnp.float32),
    )


def _invdeg_body(dg_ref, o_ref):
    o_ref[...] = 1.0 / jnp.maximum(dg_ref[0] + dg_ref[1], 1.0)


@functools.lru_cache(maxsize=None)
def _make_invdeg(n, d, bn):
    return pl.pallas_call(
        _invdeg_body,
        grid=(n // bn,),
        in_specs=[pl.BlockSpec((NC, bn, d), lambda i: (0, i, 0))],
        out_specs=pl.BlockSpec((bn, d), lambda i: (i, 0)),
        out_shape=jax.ShapeDtypeStruct((n, d), jnp.float32),
    )


def _layer_body(p_ref, dg_ref, h_ref, wl_ref, bl_ref, wr_ref, o_ref):
    ssum = p_ref[0] + p_ref[1]
    aggr = ssum * dg_ref[...]
    y = lax.dot_general(aggr, wl_ref[...], (((1,), (1,)), ((), ())),
                        preferred_element_type=jnp.float32)
    y2 = lax.dot_general(h_ref[...], wr_ref[...], (((1,), (1,)), ((), ())),
                         preferred_element_type=jnp.float32)
    o_ref[...] = jnp.maximum(y + bl_ref[...] + y2, 0.0)


@functools.lru_cache(maxsize=None)
def _make_layer(n, d, bn):
    return pl.pallas_call(
        _layer_body,
        grid=(n // bn,),
        in_specs=[
            pl.BlockSpec((NC, bn, d), lambda i: (0, i, 0)),
            pl.BlockSpec((bn, DEG_W), lambda i: (i, 0)),
            pl.BlockSpec((bn, d), lambda i: (i, 0)),
            pl.BlockSpec((d, d), lambda i: (0, 0)),
            pl.BlockSpec((1, d), lambda i: (0, 0)),
            pl.BlockSpec((d, d), lambda i: (0, 0)),
        ],
        out_specs=pl.BlockSpec((bn, d), lambda i: (i, 0)),
        out_shape=jax.ShapeDtypeStruct((n, d), jnp.float32),
    )


def kernel(x, edge_index, W_emb, b_emb,
           W_l0, b_l0, W_r0,
           W_l1, b_l1, W_r1,
           W_l2, b_l2, W_r2,
           W_l3, b_l3, W_r3):
    n, d = x.shape
    e = edge_index.shape[1]
    nch = e // (NC * NS * KAGG)
    src3 = edge_index[0].reshape(NC * NS, nch, KAGG)
    dst3 = edge_index[1].reshape(NC * NS, nch, KAGG)
    idx2 = jnp.stack([src3, dst3], axis=2)
    dstd = edge_index[1].reshape(NC * NS, e // (NC * NS * KDEG), KDEG)

    bn = 2000
    emb = _make_emb(n, d, bn)
    layer = _make_layer(n, d, bn)
    aggr = _make_aggr(n, d, e)
    deg = _make_deg(n, e)
    invdeg = _make_invdeg(n, d, bn)

    h = emb(x, W_emb, b_emb.reshape(1, d))
    degp = invdeg(deg(dstd))
    for wl, bl, wr in ((W_l0, b_l0, W_r0), (W_l1, b_l1, W_r1),
                       (W_l2, b_l2, W_r2), (W_l3, b_l3, W_r3)):
        p = aggr(h, idx2)
        h = layer(p, degp, h, wl, bl.reshape(1, d), wr)
    return h


# speedup vs baseline: 11.6447x; 1.0045x over previous
import functools

import numpy as np

import jax
import jax.numpy as jnp
from jax import lax
from jax.experimental import pallas as pl
from jax.experimental.pallas import tpu as pltpu
from jax.experimental.pallas import tpu_sc as plsc

NC = 2
NS = 16
LANES = 16
DEG_W = 128


def _fill_rows(ref, nrows, width, const):
    v = jnp.full((LANES,), const, jnp.float32)

    def body(r, _):
        for k in range(width // LANES):
            ref[r, pl.ds(k * LANES, LANES)] = v
        return 0

    lax.fori_loop(0, nrows, body, 0)


def _zero_rows(ref, nrows, width):
    _fill_rows(ref, nrows, width, 0.0)


def _copy_rows(src_at, dst_at, nrows, blk):
    for off in range(0, nrows, blk):
        n = min(blk, nrows - off)
        pltpu.sync_copy(src_at(off, n), dst_at(off, n))


def _copy_rows_async(src_at, dst_at, nrows, blk, sem):
    for off in range(0, nrows, blk):
        n = min(blk, nrows - off)
        pltpu.async_copy(src_at(off, n), dst_at(off, n), sem)
    for off in range(0, nrows, blk):
        n = min(blk, nrows - off)
        pltpu.make_async_copy(src_at(off, n), dst_at(off, n), sem).wait()


NBUF = 4
NIDX = 8
LOOK = NBUF - 1
KAGG = 40
KDEG = 80


@functools.lru_cache(maxsize=None)
def _make_aggr(n, d, e):
    ept = e // (NC * NS)
    k = KAGG
    assert ept % k == 0 and e % (NC * NS) == 0
    nch = ept // k
    rpt = (n // NS) // 8 * 8
    rem = n - rpt * NS
    assert rem % 8 == 0
    mesh = plsc.VectorSubcoreMesh(core_axis_name="c", subcore_axis_name="s")

    @functools.partial(
        pl.kernel,
        out_type=jax.ShapeDtypeStruct((NC, n, d), jnp.float32),
        mesh=mesh,
        scratch_types=[
            pltpu.VMEM_SHARED((n, d), jnp.float32),
            pltpu.VMEM((NIDX, 2, k), jnp.int32),
            pltpu.VMEM((NBUF * k, d), jnp.float32),
        ] + [pltpu.SemaphoreType.DMA] * (2 * NBUF + NIDX),
    )
    def aggr(h_hbm, idx_hbm, out_hbm, acc_sh, idx_t, rows, *sems):
        isem = sems[:NIDX]
        gsem = sems[NIDX:NIDX + NBUF]
        ssem = sems[NIDX + NBUF:]
        c = lax.axis_index("c")
        s = lax.axis_index("s")
        wid = s * NC + c
        row0 = s * rpt

        buf = [rows.at[pl.ds(b * k, k)] for b in range(NBUF)]

        def i_start(g, sl):
            pltpu.async_copy(idx_hbm.at[wid, g], idx_t.at[sl], isem[sl])

        def i_wait(g, sl):
            pltpu.make_async_copy(idx_hbm.at[wid, g], idx_t.at[sl], isem[sl]).wait()

        def g_start(b, sl):
            pltpu.async_copy(h_hbm.at[idx_t.at[sl, 0]], buf[b], gsem[b])

        def g_wait(b, sl):
            pltpu.make_async_copy(h_hbm.at[idx_t.at[sl, 0]], buf[b], gsem[b]).wait()

        def s_start(b, sl):
            pltpu.async_copy(buf[b], acc_sh.at[idx_t.at[sl, 1]], ssem[b], add=True)

        def s_wait(b, sl):
            pltpu.make_async_copy(buf[b], acc_sh.at[idx_t.at[sl, 1]], ssem[b]).wait()

        zbuf = rows.at[pl.ds(0, 128)]
        _zero_rows(rows, 128, d)
        if rem:
            @pl.when(s == NS - 1)
            def _():
                _copy_rows(lambda o, m: zbuf.at[pl.ds(0, m)],
                           lambda o, m: acc_sh.at[pl.ds(rpt * NS + o, m)],
                           rem, 128)
        _copy_rows_async(lambda o, m: zbuf.at[pl.ds(0, m)],
                         lambda o, m: acc_sh.at[pl.ds(row0 + o, m)],
                         rpt, 128, gsem[0])
        plsc.subcore_barrier()

        for sl in range(NIDX):
            i_start(sl, sl)
        for b in range(LOOK):
            i_wait(b, b)
            g_start(b, b)

        ngrp = nch // NIDX
        tail = nch - ngrp * NIDX

        def step_main(g, j):
            b, sl = j % NBUF, j % NIDX
            bn_, sln_ = (b + LOOK) % NBUF, (sl + LOOK) % NIDX
            slp_ = (sl + NIDX - 1) % NIDX
            g_wait(b, sl)

            def drain_and_prefetch():
                s_wait(bn_, slp_)

                @pl.when(g + NIDX - 1 < nch)
                def _():
                    i_start(g + NIDX - 1, slp_)
            if j == 0:
                pl.when(g > 0)(drain_and_prefetch)
            else:
                drain_and_prefetch()

            @pl.when(g + LOOK < nch)
            def _():
                i_wait(g + LOOK, sln_)
                g_start(bn_, sln_)
            s_start(b, sl)

        def group(i, _):
            for j in range(NIDX):
                step_main(i * NIDX + j, j)
            return 0

        lax.fori_loop(0, ngrp, group, 0)
        for t in range(tail):
            g = ngrp * NIDX + t
            b, sl = g % NBUF, g % NIDX
            bn_, sln_ = (b + LOOK) % NBUF, (sl + LOOK) % NIDX
            slp_ = (sl + NIDX - 1) % NIDX
            g_wait(b, sl)
            s_wait(bn_, slp_)
            if g + LOOK < nch:
                i_wait(g + LOOK, sln_)
                g_start(bn_, sln_)
            s_start(b, sl)
        s_wait((nch - 1) % NBUF, (nch - 1) % NIDX)
        plsc.subcore_barrier()

        if rem:
            @pl.when(s == NS - 1)
            def _():
                _copy_rows(lambda o, m: acc_sh.at[pl.ds(rpt * NS + o, m)],
                           lambda o, m: out_hbm.at[c, pl.ds(rpt * NS + o, m)],
                           rem, 128)
        _copy_rows_async(lambda o, m: acc_sh.at[pl.ds(row0 + o, m)],
                         lambda o, m: out_hbm.at[c, pl.ds(row0 + o, m)],
                         rpt, 128, gsem[0])

    return aggr


@functools.lru_cache(maxsize=None)
def _make_deg(n, e, deg_w=DEG_W):
    DW = deg_w
    ept = e // (NC * NS)
    k = KDEG
    assert ept % k == 0
    nch = ept // k
    rpt = (n // NS) // 8 * 8
    rem = n - rpt * NS
    assert rem % 8 == 0
    mesh = plsc.VectorSubcoreMesh(core_axis_name="c", subcore_axis_name="s")

    @functools.partial(
        pl.kernel,
        out_type=jax.ShapeDtypeStruct((NC, n, DW), jnp.float32),
        mesh=mesh,
        scratch_types=[
            pltpu.VMEM_SHARED((n, DW), jnp.float32),
            pltpu.VMEM((nch, k), jnp.int32),
            pltpu.VMEM((k, DW), jnp.float32),
            pltpu.VMEM((64, DW), jnp.float32),
        ] + [pltpu.SemaphoreType.DMA] * NBUF,
    )
    def deg(dst_hbm, out_hbm, acc_sh, dst_t, ones_v, zbuf, *ssem):
        c = lax.axis_index("c")
        s = lax.axis_index("s")
        wid = s * NC + c
        row0 = s * rpt

        def s_start(g, b):
            pltpu.async_copy(ones_v, acc_sh.at[dst_t.at[g]], ssem[b], add=True)

        def s_wait(g, b):
            pltpu.make_async_copy(ones_v, acc_sh.at[dst_t.at[g]], ssem[b]).wait()

        pltpu.sync_copy(dst_hbm.at[wid], dst_t)
        _fill_rows(ones_v, k, DW, 1.0)
        _zero_rows(zbuf, 64, DW)
        if rem:
            @pl.when(s == NS - 1)
            def _():
                _copy_rows(lambda o, m: zbuf.at[pl.ds(0, m)],
                           lambda o, m: acc_sh.at[pl.ds(rpt * NS + o, m)],
                           rem, 64)
        _copy_rows_async(lambda o, m: zbuf.at[pl.ds(0, m)],
                         lambda o, m: acc_sh.at[pl.ds(row0 + o, m)],
                         rpt, 64, ssem[0])
        plsc.subcore_barrier()

        for b in range(min(NBUF, nch)):
            s_start(b, b)
        ngrp = nch // NBUF
        tail = nch - ngrp * NBUF

        def group(i, _):
            for j in range(NBUF):
                g = i * NBUF + NBUF + j
                s_wait(g - NBUF, j)
                s_start(g, j)
            return 0

        lax.fori_loop(0, ngrp - 1, group, 0)
        for j in range(tail):
            g = (ngrp - 1) * NBUF + NBUF + j
            s_wait(g - NBUF, j)
            s_start(g, j)
        for g in range(nch - NBUF, nch):
            s_wait(g, g % NBUF)
        plsc.subcore_barrier()

        if rem:
            @pl.when(s == NS - 1)
            def _():
                _copy_rows(lambda o, m: acc_sh.at[pl.ds(rpt * NS + o, m)],
                           lambda o, m: out_hbm.at[c, pl.ds(rpt * NS + o, m)],
                           rem, 128)
        _copy_rows_async(lambda o, m: acc_sh.at[pl.ds(row0 + o, m)],
                         lambda o, m: out_hbm.at[c, pl.ds(row0 + o, m)],
                         rpt, 128, ssem[0])

    return deg


def _emb_body(x_ref, w_ref, b_ref, o_ref):
    y = lax.dot_general(x_ref[...], w_ref[...], (((1,), (1,)), ((), ())),
                        preferred_element_type=jnp.float32)
    o_ref[...] = jnp.maximum(y + b_ref[...], 0.0)


@functools.lru_cache(maxsize=None)
def _make_emb(n, d, bn):
    return pl.pallas_call(
        _emb_body,
        grid=(n // bn,),
        in_specs=[
            pl.BlockSpec((bn, d), lambda i: (i, 0)),
            pl.BlockSpec((d, d), lambda i: (0, 0)),
            pl.BlockSpec((1, d), lambda i: (0, 0)),
        ],
        out_specs=pl.BlockSpec((bn, d), lambda i: (i, 0)),
        out_shape=jax.ShapeDtypeStruct((n, d), jnp.float32),
    )


def _invdeg_body(dg_ref, o_ref):
    o_ref[...] = 1.0 / jnp.maximum(dg_ref[0] + dg_ref[1], 1.0)


@functools.lru_cache(maxsize=None)
def _make_invdeg(n, d, bn):
    return pl.pallas_call(
        _invdeg_body,
        grid=(n // bn,),
        in_specs=[pl.BlockSpec((NC, bn, d), lambda i: (0, i, 0))],
        out_specs=pl.BlockSpec((bn, d), lambda i: (i, 0)),
        out_shape=jax.ShapeDtypeStruct((n, d), jnp.float32),
    )


def _layer_body(p_ref, dg_ref, h_ref, wl_ref, bl_ref, wr_ref, o_ref):
    ssum = p_ref[0] + p_ref[1]
    aggr = ssum * dg_ref[...]
    y = lax.dot_general(aggr, wl_ref[...], (((1,), (1,)), ((), ())),
                        preferred_element_type=jnp.float32)
    y2 = lax.dot_general(h_ref[...], wr_ref[...], (((1,), (1,)), ((), ())),
                         preferred_element_type=jnp.float32)
    o_ref[...] = jnp.maximum(y + bl_ref[...] + y2, 0.0)


@functools.lru_cache(maxsize=None)
def _make_layer(n, d, bn):
    return pl.pallas_call(
        _layer_body,
        grid=(n // bn,),
        in_specs=[
            pl.BlockSpec((NC, bn, d), lambda i: (0, i, 0)),
            pl.BlockSpec((bn, DEG_W), lambda i: (i, 0)),
            pl.BlockSpec((bn, d), lambda i: (i, 0)),
            pl.BlockSpec((d, d), lambda i: (0, 0)),
            pl.BlockSpec((1, d), lambda i: (0, 0)),
            pl.BlockSpec((d, d), lambda i: (0, 0)),
        ],
        out_specs=pl.BlockSpec((bn, d), lambda i: (i, 0)),
        out_shape=jax.ShapeDtypeStruct((n, d), jnp.float32),
    )


def kernel(x, edge_index, W_emb, b_emb,
           W_l0, b_l0, W_r0,
           W_l1, b_l1, W_r1,
           W_l2, b_l2, W_r2,
           W_l3, b_l3, W_r3):
    n, d = x.shape
    e = edge_index.shape[1]
    nch = e // (NC * NS * KAGG)
    src3 = edge_index[0].reshape(NC * NS, nch, KAGG)
    dst3 = edge_index[1].reshape(NC * NS, nch, KAGG)
    idx2 = jnp.stack([src3, dst3], axis=2)
    dstd = edge_index[1].reshape(NC * NS, e // (NC * NS * KDEG), KDEG)

    bn = 2000
    emb = _make_emb(n, d, bn)
    layer = _make_layer(n, d, bn)
    aggr = _make_aggr(n, d, e)
    deg = _make_deg(n, e)
    invdeg = _make_invdeg(n, d, bn)

    h = emb(x, W_emb, b_emb.reshape(1, d))
    degp = invdeg(deg(dstd))
    for wl, bl, wr in ((W_l0, b_l0, W_r0), (W_l1, b_l1, W_r1),
                       (W_l2, b_l2, W_r2), (W_l3, b_l3, W_r3)):
        p = aggr(h, idx2)
        h = layer(p, degp, h, wl, bl.reshape(1, d), wr)
    return h
